# Initial kernel scaffold; baseline (speedup 1.0000x reference)
import jax
import jax.numpy as jnp
from jax import lax
from jax.experimental import pallas as pl
from jax.experimental.pallas import tpu as pltpu

_N = 20000
_PAD = 20480
_ROWS = _PAD // 128
_MAX_OUT = 300
_IOU_THRESH = 0.5
_NEG = jnp.float32(-1e30)


def _nms_body(b0, b1, b2, b3, s, out, x1s, y1s, x2s, y2s, ars, ms):
    x1 = jnp.minimum(b0[:], b2[:])
    y1 = jnp.minimum(b1[:], b3[:])
    x2 = jnp.maximum(b0[:], b2[:])
    y2 = jnp.maximum(b1[:], b3[:])
    x1s[:] = x1
    y1s[:] = y1
    x2s[:] = x2
    y2s[:] = y2
    ars[:] = (x2 - x1) * (y2 - y1)
    ms[:] = s[:]
    out[:] = jnp.zeros_like(out)

    row_i = lax.broadcasted_iota(jnp.int32, (_ROWS, 128), 0)
    lane_i = lax.broadcasted_iota(jnp.int32, (_ROWS, 128), 1)
    idx2d = row_i * 128 + lane_i
    lane1 = lax.broadcasted_iota(jnp.int32, (1, 128), 1)

    def step(i, _):
        msv = ms[:]
        m = jnp.max(msv)
        idx = jnp.min(jnp.where(msv == m, idx2d, jnp.int32(2**30)))
        valid = m > _NEG / 2
        r = idx // 128
        c = idx % 128
        onehot = (lane1 == c).astype(jnp.float32)
        sx1 = jnp.sum(x1s[pl.ds(r, 1), :] * onehot)
        sy1 = jnp.sum(y1s[pl.ds(r, 1), :] * onehot)
        sx2 = jnp.sum(x2s[pl.ds(r, 1), :] * onehot)
        sy2 = jnp.sum(y2s[pl.ds(r, 1), :] * onehot)
        sar = jnp.sum(ars[pl.ds(r, 1), :] * onehot)

        iw = jnp.maximum(jnp.minimum(x2s[:], sx2) - jnp.maximum(x1s[:], sx1), 0.0)
        ih = jnp.maximum(jnp.minimum(y2s[:], sy2) - jnp.maximum(y1s[:], sy1), 0.0)
        inter = iw * ih
        iou = inter / (ars[:] + sar - inter + jnp.float32(1e-8))
        keep = (iou <= _IOU_THRESH) & (idx2d != idx)
        ms[:] = jnp.where(keep, msv, _NEG)

        vf = jnp.where(valid, jnp.float32(1.0), jnp.float32(0.0))
        rowv = (
            jnp.where(lane1 == 0, sx1, 0.0)
            + jnp.where(lane1 == 1, sy1, 0.0)
            + jnp.where(lane1 == 2, sx2, 0.0)
            + jnp.where(lane1 == 3, sy2, 0.0)
            + jnp.where(lane1 == 4, m, 0.0)
        ) * vf
        out[pl.ds(i, 1), :] = rowv
        return 0

    lax.fori_loop(0, _MAX_OUT, step, 0)


def _run_nms(b0, b1, b2, b3, sp):
    return pl.pallas_call(
        _nms_body,
        out_shape=jax.ShapeDtypeStruct((_MAX_OUT + 4, 128), jnp.float32),
        scratch_shapes=[pltpu.VMEM((_ROWS, 128), jnp.float32)] * 6,
    )(b0, b1, b2, b3, sp)


def kernel(boxes, scores):
    bp = jnp.pad(boxes, ((0, _PAD - _N), (0, 0)))
    sp = jnp.pad(scores, (0, _PAD - _N), constant_values=_NEG)
    b0 = bp[:, 0].reshape(_ROWS, 128)
    b1 = bp[:, 1].reshape(_ROWS, 128)
    b2 = bp[:, 2].reshape(_ROWS, 128)
    b3 = bp[:, 3].reshape(_ROWS, 128)
    out = _run_nms(b0, b1, b2, b3, sp.reshape(_ROWS, 128))
    return out[:_MAX_OUT, :5]


# speedup vs baseline: 24.5461x; 24.5461x over previous
import jax
import jax.numpy as jnp
from jax import lax
from jax.experimental import pallas as pl
from jax.experimental.pallas import tpu as pltpu

_N = 20000
_PAD = 20480
_ROWS = _PAD // 128
_MAX_OUT = 300
_IOU_THRESH = 0.5
_NEG = -1e30


def _nms_body(b0, b1, b2, b3, s, out, x1s, y1s, x2s, y2s, ars, ms):
    x1 = jnp.minimum(b0[:], b2[:])
    y1 = jnp.minimum(b1[:], b3[:])
    x2 = jnp.maximum(b0[:], b2[:])
    y2 = jnp.maximum(b1[:], b3[:])
    x1s[:] = x1
    y1s[:] = y1
    x2s[:] = x2
    y2s[:] = y2
    ars[:] = (x2 - x1) * (y2 - y1)
    ms[:] = s[:]
    out[:] = jnp.zeros_like(out)

    row_i = lax.broadcasted_iota(jnp.int32, (_ROWS, 128), 0)
    lane_i = lax.broadcasted_iota(jnp.int32, (_ROWS, 128), 1)
    idx2d = row_i * 128 + lane_i
    lane1 = lax.broadcasted_iota(jnp.int32, (1, 128), 1)

    def step(i, _):
        msv = ms[:]
        m = jnp.max(msv)
        idx = jnp.min(jnp.where(msv == m, idx2d, jnp.int32(2**30)))
        valid = m > jnp.float32(_NEG / 2)
        r = idx // 128
        c = idx % 128
        onehot = (lane1 == c).astype(jnp.float32)
        sx1 = jnp.sum(x1s[pl.ds(r, 1), :] * onehot)
        sy1 = jnp.sum(y1s[pl.ds(r, 1), :] * onehot)
        sx2 = jnp.sum(x2s[pl.ds(r, 1), :] * onehot)
        sy2 = jnp.sum(y2s[pl.ds(r, 1), :] * onehot)
        sar = jnp.sum(ars[pl.ds(r, 1), :] * onehot)

        iw = jnp.maximum(jnp.minimum(x2s[:], sx2) - jnp.maximum(x1s[:], sx1), 0.0)
        ih = jnp.maximum(jnp.minimum(y2s[:], sy2) - jnp.maximum(y1s[:], sy1), 0.0)
        inter = iw * ih
        iou = inter / (ars[:] + sar - inter + jnp.float32(1e-8))
        keep = (iou <= _IOU_THRESH) & (idx2d != idx)
        ms[:] = jnp.where(keep, msv, jnp.float32(_NEG))

        vf = jnp.where(valid, jnp.float32(1.0), jnp.float32(0.0))
        rowv = (
            jnp.where(lane1 == 0, sx1, 0.0)
            + jnp.where(lane1 == 1, sy1, 0.0)
            + jnp.where(lane1 == 2, sx2, 0.0)
            + jnp.where(lane1 == 3, sy2, 0.0)
            + jnp.where(lane1 == 4, m, 0.0)
        ) * vf
        out[pl.ds(i, 1), :] = rowv
        return 0

    lax.fori_loop(0, _MAX_OUT, step, 0)


def _run_nms(b0, b1, b2, b3, sp):
    return pl.pallas_call(
        _nms_body,
        out_shape=jax.ShapeDtypeStruct((_MAX_OUT + 4, 128), jnp.float32),
        scratch_shapes=[pltpu.VMEM((_ROWS, 128), jnp.float32)] * 6,
    )(b0, b1, b2, b3, sp)


def kernel(boxes, scores):
    bp = jnp.pad(boxes, ((0, _PAD - _N), (0, 0)))
    sp = jnp.pad(scores, (0, _PAD - _N), constant_values=-1e30)
    b0 = bp[:, 0].reshape(_ROWS, 128)
    b1 = bp[:, 1].reshape(_ROWS, 128)
    b2 = bp[:, 2].reshape(_ROWS, 128)
    b3 = bp[:, 3].reshape(_ROWS, 128)
    out = _run_nms(b0, b1, b2, b3, sp.reshape(_ROWS, 128))
    return out[:_MAX_OUT, :5]
